# Initial kernel scaffold; baseline (speedup 1.0000x reference)
import jax
import jax.numpy as jnp
from jax.experimental import pallas as pl


def kernel(query, reference_points, value_0, value_1, value_2, W_value, b_value, W_off, b_off, W_attn, b_attn, W_out, b_out):
    raise NotImplementedError("write your pallas kernel here")


# speedup vs baseline: 8.7504x; 8.7504x over previous
import functools

import jax
import jax.numpy as jnp
from jax import lax
from jax.experimental import pallas as pl
from jax.experimental.pallas import tpu as pltpu
from jax.experimental.pallas import tpu_sc as plsc

D = 256
NH = 8
NL = 3
NP = 4
HD = D // NH
SPATIAL = ((64, 64), (32, 32), (16, 16))
LVL_BASE = (0, 4096, 5120)
TOT_POS = 5376
BB = 4
NQ = 1024
BQ = BB * NQ
S = BQ * NH * NL * NP
S4 = S * 4
NSLOT = BQ * NH
TABLE_ROWS = BB * TOT_POS * NH

NW = 32
SLOTS_PER_W = NSLOT // NW
CH_SLOTS = 8
CH_ROWS = CH_SLOTS * 48
CH_IDXROWS = CH_ROWS // 128
N_CHUNK = SLOTS_PER_W // CH_SLOTS


def _mm_bias(x, w, b, bm):
    M, K = x.shape
    N = w.shape[1]

    def body(x_ref, w_ref, b_ref, o_ref):
        o_ref[...] = (
            jnp.dot(x_ref[...], w_ref[...], preferred_element_type=jnp.float32)
            + b_ref[...]
        )

    return pl.pallas_call(
        body,
        grid=(M // bm,),
        in_specs=[
            pl.BlockSpec((bm, K), lambda i: (i, 0)),
            pl.BlockSpec((K, N), lambda i: (0, 0)),
            pl.BlockSpec((1, N), lambda i: (0, 0)),
        ],
        out_specs=pl.BlockSpec((bm, N), lambda i: (i, 0)),
        out_shape=jax.ShapeDtypeStruct((M, N), jnp.float32),
    )(x, w, b.reshape(1, N))


def _qproj(q2d, w_off, b_off, w_attn, b_attn, bd):
    M = q2d.shape[0]
    bm = 512

    def body(q_ref, wo_ref, bo_ref, wa_ref, ba_ref, bd_ref, off_ref, aw_ref):
        q = q_ref[...]
        off_ref[...] = (
            jnp.dot(q, wo_ref[...], preferred_element_type=jnp.float32)
            + bo_ref[...]
        )
        lg = (
            jnp.dot(q, wa_ref[...], preferred_element_type=jnp.float32)
            + ba_ref[...]
        )
        m = jnp.exp(lg - jnp.max(lg, axis=-1, keepdims=True))
        denom = jnp.dot(m, bd_ref[...], preferred_element_type=jnp.float32)
        aw_ref[...] = m / denom

    n_off = w_off.shape[1]
    n_at = w_attn.shape[1]
    return pl.pallas_call(
        body,
        grid=(M // bm,),
        in_specs=[
            pl.BlockSpec((bm, D), lambda i: (i, 0)),
            pl.BlockSpec((D, n_off), lambda i: (0, 0)),
            pl.BlockSpec((1, n_off), lambda i: (0, 0)),
            pl.BlockSpec((D, n_at), lambda i: (0, 0)),
            pl.BlockSpec((1, n_at), lambda i: (0, 0)),
            pl.BlockSpec((n_at, n_at), lambda i: (0, 0)),
        ],
        out_specs=[
            pl.BlockSpec((bm, n_off), lambda i: (i, 0)),
            pl.BlockSpec((bm, n_at), lambda i: (i, 0)),
        ],
        out_shape=[
            jax.ShapeDtypeStruct((M, n_off), jnp.float32),
            jax.ShapeDtypeStruct((M, n_at), jnp.float32),
        ],
    )(q2d, w_off, b_off.reshape(1, n_off), w_attn, b_attn.reshape(1, n_at), bd)


def _samp_prep(refx, refy, offx, offy, aw, wf, hf, ibase):
    SR = S // 128
    bm = 512

    def body(rx_ref, ry_ref, ox_ref, oy_ref, aw_ref, wf_ref, hf_ref, ib_ref,
             i00, i10, i01, i11, w00, w10, w01, w11):
        wfv = wf_ref[...]
        hfv = hf_ref[...]
        ix = rx_ref[...] * wfv + ox_ref[...] - 0.5
        iy = ry_ref[...] * hfv + oy_ref[...] - 0.5
        x0f = jnp.floor(ix)
        y0f = jnp.floor(iy)
        wx1 = ix - x0f
        wx0 = 1.0 - wx1
        wy1 = iy - y0f
        wy0 = 1.0 - wy1
        x0 = x0f.astype(jnp.int32)
        y0 = y0f.astype(jnp.int32)
        x1 = x0 + 1
        y1 = y0 + 1
        wi = wfv.astype(jnp.int32)
        hi = hfv.astype(jnp.int32)
        ib = ib_ref[...]
        awv = aw_ref[...]

        def corner(xi, yi, wx, wy, iref, wref):
            valid = (xi >= 0) & (xi < wi) & (yi >= 0) & (yi < hi)
            xc = jnp.clip(xi, 0, wi - 1)
            yc = jnp.clip(yi, 0, hi - 1)
            iref[...] = ib + yc * (wi * 8) + xc * 8
            wref[...] = jnp.where(valid, awv * wx * wy, 0.0)

        corner(x0, y0, wx0, wy0, i00, w00)
        corner(x1, y0, wx1, wy0, i10, w10)
        corner(x0, y1, wx0, wy1, i01, w01)
        corner(x1, y1, wx1, wy1, i11, w11)

    spec = pl.BlockSpec((bm, 128), lambda i: (i, 0))
    f32 = jnp.float32
    return pl.pallas_call(
        body,
        grid=(SR // bm,),
        in_specs=[spec] * 8,
        out_specs=[spec] * 8,
        out_shape=[jax.ShapeDtypeStruct((SR, 128), jnp.int32)] * 4
        + [jax.ShapeDtypeStruct((SR, 128), f32)] * 4,
    )(
        refx.reshape(SR, 128), refy.reshape(SR, 128),
        offx.reshape(SR, 128), offy.reshape(SR, 128),
        aw.reshape(SR, 128), wf.reshape(SR, 128), hf.reshape(SR, 128),
        ibase.reshape(SR, 128),
    )


def _sc_gather_combine(table, idx2d, wts):
    mesh = plsc.VectorSubcoreMesh(
        core_axis_name="c", subcore_axis_name="s", num_cores=2, num_subcores=16
    )
    nc = 2

    @functools.partial(
        pl.kernel,
        out_type=jax.ShapeDtypeStruct((NSLOT, HD), jnp.float32),
        mesh=mesh,
        scratch_types=[
            pltpu.VMEM((CH_ROWS,), jnp.int32),
            pltpu.VMEM((CH_ROWS,), jnp.float32),
            pltpu.VMEM((CH_ROWS, HD), jnp.float32),
            pltpu.VMEM((CH_SLOTS, HD), jnp.float32),
            pltpu.SemaphoreType.DMA,
        ],
        compiler_params=pltpu.CompilerParams(use_tc_tiling_on_sc=False),
    )
    def k(table_hbm, idx_hbm, w_hbm, out_hbm, idx_v, w_v, rows_v, out_v, sem):
        wid = lax.axis_index("s") * nc + lax.axis_index("c")
        slot0 = wid * SLOTS_PER_W

        def chunk(c, _):
            slot = slot0 + c * CH_SLOTS
            row0 = slot * 48
            pltpu.sync_copy(idx_hbm.at[pl.ds(row0, CH_ROWS)], idx_v)
            pltpu.sync_copy(w_hbm.at[pl.ds(row0, CH_ROWS)], w_v)
            cps = [
                pltpu.async_copy(
                    table_hbm.at[idx_v.at[pl.ds(j * 128, 128)]],
                    rows_v.at[pl.ds(j * 128, 128)],
                    sem,
                )
                for j in range(CH_IDXROWS)
            ]
            for cp in cps:
                cp.wait()
            dn = lax.GatherDimensionNumbers(
                offset_dims=(), collapsed_slice_dims=(0,), start_index_map=(0,)
            )
            for g in range(CH_SLOTS):
                acc0 = jnp.zeros((16,), jnp.float32)
                acc1 = jnp.zeros((16,), jnp.float32)
                for t3 in range(3):
                    base = g * 48 + t3 * 16
                    wgrp = w_v[pl.ds(base, 16)]
                    for t in range(16):
                        r = base + t
                        wb = lax.gather(
                            wgrp,
                            jnp.full((16, 1), t, jnp.int32),
                            dn,
                            (1,),
                            mode=lax.GatherScatterMode.PROMISE_IN_BOUNDS,
                        )
                        acc0 = acc0 + wb * rows_v[r, pl.ds(0, 16)]
                        acc1 = acc1 + wb * rows_v[r, pl.ds(16, 16)]
                out_v[g, pl.ds(0, 16)] = acc0
                out_v[g, pl.ds(16, 16)] = acc1
            pltpu.sync_copy(out_v, out_hbm.at[pl.ds(slot, CH_SLOTS)])
            return 0

        lax.fori_loop(0, N_CHUNK, chunk, 0)

    return k(table, idx2d, wts)


def kernel(query, reference_points, value_0, value_1, value_2, W_value,
           b_value, W_off, b_off, W_attn, b_attn, W_out, b_out):
    f32 = jnp.float32
    v_all = jnp.concatenate([value_0, value_1, value_2], axis=1)
    v_all = v_all.reshape(BB * TOT_POS, D)
    table = _mm_bias(v_all, W_value, b_value, bm=256)
    table = table.reshape(TABLE_ROWS, HD)

    q2d = query.reshape(BQ, D)
    n_at = NH * NL * NP
    seg = jnp.arange(n_at, dtype=jnp.int32) // (NL * NP)
    bd = (seg[:, None] == seg[None, :]).astype(f32)
    off2d, aw2d = _qproj(q2d, W_off, b_off, W_attn, b_attn, bd)

    off = off2d.reshape(BQ, NH, NL, NP, 2)
    offx = off[..., 0].reshape(S)
    offy = off[..., 1].reshape(S)
    aw = aw2d.reshape(S)
    rp = reference_points.reshape(BQ, NL, 2)
    refx = jnp.broadcast_to(rp[:, None, :, None, 0], (BQ, NH, NL, NP)).reshape(S)
    refy = jnp.broadcast_to(rp[:, None, :, None, 1], (BQ, NH, NL, NP)).reshape(S)

    wl = jnp.array([w for (_, w) in SPATIAL], f32)
    hl = jnp.array([h for (h, _) in SPATIAL], f32)
    wf = jnp.broadcast_to(wl[None, None, :, None], (BQ, NH, NL, NP)).reshape(S)
    hf = jnp.broadcast_to(hl[None, None, :, None], (BQ, NH, NL, NP)).reshape(S)
    b_ix = jnp.arange(BQ, dtype=jnp.int32) // NQ
    h_ix = jnp.arange(NH, dtype=jnp.int32)
    base_l = jnp.array(LVL_BASE, jnp.int32)
    ibase = (
        (b_ix[:, None, None] * TOT_POS + base_l[None, None, :]) * 8
        + h_ix[None, :, None]
    )
    ibase = jnp.broadcast_to(ibase[..., None], (BQ, NH, NL, NP)).reshape(S)

    i00, i10, i01, i11, w00, w10, w01, w11 = _samp_prep(
        refx, refy, offx, offy, aw, wf, hf, ibase
    )

    idx = jnp.stack(
        [x.reshape(S) for x in (i00, i10, i01, i11)], axis=-1
    ).reshape(S4)
    wts = jnp.stack(
        [x.reshape(S) for x in (w00, w10, w01, w11)], axis=-1
    ).reshape(S4)

    heads = _sc_gather_combine(table, idx, wts)

    out = _mm_bias(heads.reshape(BQ, D), W_out, b_out, bm=512)
    return out.reshape(BB, NQ, D)


# speedup vs baseline: 16.6783x; 1.9060x over previous
import functools

import jax
import jax.numpy as jnp
from jax import lax
from jax.experimental import pallas as pl
from jax.experimental.pallas import tpu as pltpu
from jax.experimental.pallas import tpu_sc as plsc

D = 256
NH = 8
NL = 3
NP = 4
HD = D // NH
SPATIAL = ((64, 64), (32, 32), (16, 16))
LVL_BASE = (0, 4096, 5120)
TOT_POS = 5376
BB = 4
NQ = 1024
BQ = BB * NQ
S = BQ * NH * NL * NP
S4 = S * 4
NSLOT = BQ * NH
TABLE_ROWS = BB * TOT_POS * NH

NW = 32
SLOTS_PER_W = NSLOT // NW
CH_SLOTS = 8
CH_ROWS = CH_SLOTS * 48
CH_IDXROWS = CH_ROWS // 128
N_CHUNK = SLOTS_PER_W // CH_SLOTS


def _mm_bias(x, w, b, bm):
    M, K = x.shape
    N = w.shape[1]

    def body(x_ref, w_ref, b_ref, o_ref):
        o_ref[...] = (
            jnp.dot(x_ref[...], w_ref[...], preferred_element_type=jnp.float32)
            + b_ref[...]
        )

    return pl.pallas_call(
        body,
        grid=(M // bm,),
        in_specs=[
            pl.BlockSpec((bm, K), lambda i: (i, 0)),
            pl.BlockSpec((K, N), lambda i: (0, 0)),
            pl.BlockSpec((1, N), lambda i: (0, 0)),
        ],
        out_specs=pl.BlockSpec((bm, N), lambda i: (i, 0)),
        out_shape=jax.ShapeDtypeStruct((M, N), jnp.float32),
    )(x, w, b.reshape(1, N))


def _qproj(q2d, w_off, b_off, w_attn, b_attn, bd):
    M = q2d.shape[0]
    bm = 512

    def body(q_ref, wo_ref, bo_ref, wa_ref, ba_ref, bd_ref, off_ref, aw_ref):
        q = q_ref[...]
        off_ref[...] = (
            jnp.dot(q, wo_ref[...], preferred_element_type=jnp.float32)
            + bo_ref[...]
        )
        lg = (
            jnp.dot(q, wa_ref[...], preferred_element_type=jnp.float32)
            + ba_ref[...]
        )
        m = jnp.exp(lg - jnp.max(lg, axis=-1, keepdims=True))
        denom = jnp.dot(m, bd_ref[...], preferred_element_type=jnp.float32)
        aw_ref[...] = m / denom

    n_off = w_off.shape[1]
    n_at = w_attn.shape[1]
    return pl.pallas_call(
        body,
        grid=(M // bm,),
        in_specs=[
            pl.BlockSpec((bm, D), lambda i: (i, 0)),
            pl.BlockSpec((D, n_off), lambda i: (0, 0)),
            pl.BlockSpec((1, n_off), lambda i: (0, 0)),
            pl.BlockSpec((D, n_at), lambda i: (0, 0)),
            pl.BlockSpec((1, n_at), lambda i: (0, 0)),
            pl.BlockSpec((n_at, n_at), lambda i: (0, 0)),
        ],
        out_specs=[
            pl.BlockSpec((bm, n_off), lambda i: (i, 0)),
            pl.BlockSpec((bm, n_at), lambda i: (i, 0)),
        ],
        out_shape=[
            jax.ShapeDtypeStruct((M, n_off), jnp.float32),
            jax.ShapeDtypeStruct((M, n_at), jnp.float32),
        ],
    )(q2d, w_off, b_off.reshape(1, n_off), w_attn, b_attn.reshape(1, n_at), bd)


def _samp_prep(refx, refy, offx, offy, aw, wf, hf, ibase):
    SR = S // 128
    bm = 512

    def body(rx_ref, ry_ref, ox_ref, oy_ref, aw_ref, wf_ref, hf_ref, ib_ref,
             i00, i10, i01, i11, w00, w10, w01, w11):
        wfv = wf_ref[...]
        hfv = hf_ref[...]
        ix = rx_ref[...] * wfv + ox_ref[...] - 0.5
        iy = ry_ref[...] * hfv + oy_ref[...] - 0.5
        x0f = jnp.floor(ix)
        y0f = jnp.floor(iy)
        wx1 = ix - x0f
        wx0 = 1.0 - wx1
        wy1 = iy - y0f
        wy0 = 1.0 - wy1
        x0 = x0f.astype(jnp.int32)
        y0 = y0f.astype(jnp.int32)
        x1 = x0 + 1
        y1 = y0 + 1
        wi = wfv.astype(jnp.int32)
        hi = hfv.astype(jnp.int32)
        ib = ib_ref[...]
        awv = aw_ref[...]

        def corner(xi, yi, wx, wy, iref, wref):
            valid = (xi >= 0) & (xi < wi) & (yi >= 0) & (yi < hi)
            xc = jnp.clip(xi, 0, wi - 1)
            yc = jnp.clip(yi, 0, hi - 1)
            iref[...] = ib + yc * (wi * 8) + xc * 8
            wref[...] = jnp.where(valid, awv * wx * wy, 0.0)

        corner(x0, y0, wx0, wy0, i00, w00)
        corner(x1, y0, wx1, wy0, i10, w10)
        corner(x0, y1, wx0, wy1, i01, w01)
        corner(x1, y1, wx1, wy1, i11, w11)

    spec = pl.BlockSpec((bm, 128), lambda i: (i, 0))
    f32 = jnp.float32
    return pl.pallas_call(
        body,
        grid=(SR // bm,),
        in_specs=[spec] * 8,
        out_specs=[spec] * 8,
        out_shape=[jax.ShapeDtypeStruct((SR, 128), jnp.int32)] * 4
        + [jax.ShapeDtypeStruct((SR, 128), f32)] * 4,
    )(
        refx.reshape(SR, 128), refy.reshape(SR, 128),
        offx.reshape(SR, 128), offy.reshape(SR, 128),
        aw.reshape(SR, 128), wf.reshape(SR, 128), hf.reshape(SR, 128),
        ibase.reshape(SR, 128),
    )


def _sc_gather_combine(table, idxs, wtss):
    mesh = plsc.VectorSubcoreMesh(
        core_axis_name="c", subcore_axis_name="s", num_cores=2, num_subcores=16
    )
    nc = 2
    cs = CH_SLOTS
    cr = cs * 12

    @functools.partial(
        pl.kernel,
        out_type=jax.ShapeDtypeStruct((NSLOT * HD,), jnp.float32),
        mesh=mesh,
        scratch_types=[
            pltpu.VMEM((4 * cr,), jnp.int32),
            pltpu.VMEM((4 * cr,), jnp.float32),
            pltpu.VMEM((4 * cr, HD), jnp.float32),
            pltpu.VMEM((cs * HD,), jnp.float32),
            pltpu.SemaphoreType.DMA,
            pltpu.SemaphoreType.DMA,
        ],
        compiler_params=pltpu.CompilerParams(use_tc_tiling_on_sc=False),
    )
    def k(table_hbm, i0, i1, i2, i3, v0, v1, v2, v3, out_hbm,
          idx_v, w_v, rows_v, out_v, semi, sem):
        wid = lax.axis_index("s") * nc + lax.axis_index("c")
        slot0 = wid * SLOTS_PER_W
        ihbm = (i0, i1, i2, i3)
        whbm = (v0, v1, v2, v3)
        dn = lax.GatherDimensionNumbers(
            offset_dims=(), collapsed_slice_dims=(0,), start_index_map=(0,)
        )

        def chunk(c, _):
            slot = slot0 + c * cs
            row0 = slot * 12
            cps_in = []
            for ci in range(4):
                cps_in.append(pltpu.async_copy(
                    ihbm[ci].at[pl.ds(row0, cr)],
                    idx_v.at[pl.ds(ci * cr, cr)], semi))
                cps_in.append(pltpu.async_copy(
                    whbm[ci].at[pl.ds(row0, cr)],
                    w_v.at[pl.ds(ci * cr, cr)], semi))
            for cp in cps_in:
                cp.wait()
            cps = [
                pltpu.async_copy(
                    table_hbm.at[idx_v.at[pl.ds(ci * cr, cr)]],
                    rows_v.at[pl.ds(ci * cr, cr)],
                    sem,
                )
                for ci in range(4)
            ]
            for cp in cps:
                cp.wait()
            accs = [
                [jnp.zeros((16,), jnp.float32), jnp.zeros((16,), jnp.float32)]
                for _ in range(cs)
            ]
            for ci in range(4):
                for grp in range(cr // 16):
                    wgrp = w_v[pl.ds(ci * cr + grp * 16, 16)]
                    for t in range(16):
                        jj = grp * 16 + t
                        g = jj // 12
                        r = ci * cr + jj
                        wb = lax.gather(
                            wgrp,
                            jnp.full((16, 1), t, jnp.int32),
                            dn,
                            (1,),
                            mode=lax.GatherScatterMode.PROMISE_IN_BOUNDS,
                        )
                        accs[g][0] = accs[g][0] + wb * rows_v[r, pl.ds(0, 16)]
                        accs[g][1] = accs[g][1] + wb * rows_v[r, pl.ds(16, 16)]
            for g in range(cs):
                out_v[pl.ds(g * HD, 16)] = accs[g][0]
                out_v[pl.ds(g * HD + 16, 16)] = accs[g][1]
            pltpu.sync_copy(out_v, out_hbm.at[pl.ds(slot * HD, cs * HD)])
            return 0

        lax.fori_loop(0, N_CHUNK, chunk, 0)

    return k(table, *idxs, *wtss)


def kernel(query, reference_points, value_0, value_1, value_2, W_value,
           b_value, W_off, b_off, W_attn, b_attn, W_out, b_out):
    f32 = jnp.float32
    v_all = jnp.concatenate([value_0, value_1, value_2], axis=1)
    v_all = v_all.reshape(BB * TOT_POS, D)
    table = _mm_bias(v_all, W_value, b_value, bm=256)
    table = table.reshape(TABLE_ROWS, HD)

    q2d = query.reshape(BQ, D)
    n_at = NH * NL * NP
    seg = jnp.arange(n_at, dtype=jnp.int32) // (NL * NP)
    bd = (seg[:, None] == seg[None, :]).astype(f32)
    off2d, aw2d = _qproj(q2d, W_off, b_off, W_attn, b_attn, bd)

    off = off2d.reshape(BQ, NH, NL, NP, 2)
    offx = off[..., 0].reshape(S)
    offy = off[..., 1].reshape(S)
    aw = aw2d.reshape(S)
    rp = reference_points.reshape(BQ, NL, 2)
    refx = jnp.broadcast_to(rp[:, None, :, None, 0], (BQ, NH, NL, NP)).reshape(S)
    refy = jnp.broadcast_to(rp[:, None, :, None, 1], (BQ, NH, NL, NP)).reshape(S)

    wl = jnp.array([w for (_, w) in SPATIAL], f32)
    hl = jnp.array([h for (h, _) in SPATIAL], f32)
    wf = jnp.broadcast_to(wl[None, None, :, None], (BQ, NH, NL, NP)).reshape(S)
    hf = jnp.broadcast_to(hl[None, None, :, None], (BQ, NH, NL, NP)).reshape(S)
    b_ix = jnp.arange(BQ, dtype=jnp.int32) // NQ
    h_ix = jnp.arange(NH, dtype=jnp.int32)
    base_l = jnp.array(LVL_BASE, jnp.int32)
    ibase = (
        (b_ix[:, None, None] * TOT_POS + base_l[None, None, :]) * 8
        + h_ix[None, :, None]
    )
    ibase = jnp.broadcast_to(ibase[..., None], (BQ, NH, NL, NP)).reshape(S)

    i00, i10, i01, i11, w00, w10, w01, w11 = _samp_prep(
        refx, refy, offx, offy, aw, wf, hf, ibase
    )

    heads = _sc_gather_combine(
        table,
        [x.reshape(S) for x in (i00, i10, i01, i11)],
        [x.reshape(S) for x in (w00, w10, w01, w11)],
    )

    out = _mm_bias(heads.reshape(BQ, D), W_out, b_out, bm=512)
    return out.reshape(BB, NQ, D)


# speedup vs baseline: 19.3723x; 1.1615x over previous
import functools

import jax
import jax.numpy as jnp
from jax import lax
from jax.experimental import pallas as pl
from jax.experimental.pallas import tpu as pltpu
from jax.experimental.pallas import tpu_sc as plsc

D = 256
NH = 8
NL = 3
NP = 4
HD = D // NH
SPATIAL = ((64, 64), (32, 32), (16, 16))
LVL_BASE = (0, 4096, 5120)
TOT_POS = 5376
BB = 4
NQ = 1024
BQ = BB * NQ
S = BQ * NH * NL * NP
S4 = S * 4
NSLOT = BQ * NH
TABLE_ROWS = BB * TOT_POS * NH

NW = 32
SLOTS_PER_W = NSLOT // NW
CH_SLOTS = 8
CH_ROWS = CH_SLOTS * 48
CH_IDXROWS = CH_ROWS // 128
N_CHUNK = SLOTS_PER_W // CH_SLOTS


def _mm_bias(x, w, b, bm):
    M, K = x.shape
    N = w.shape[1]

    def body(x_ref, w_ref, b_ref, o_ref):
        o_ref[...] = (
            jnp.dot(x_ref[...], w_ref[...], preferred_element_type=jnp.float32)
            + b_ref[...]
        )

    return pl.pallas_call(
        body,
        grid=(M // bm,),
        in_specs=[
            pl.BlockSpec((bm, K), lambda i: (i, 0)),
            pl.BlockSpec((K, N), lambda i: (0, 0)),
            pl.BlockSpec((1, N), lambda i: (0, 0)),
        ],
        out_specs=pl.BlockSpec((bm, N), lambda i: (i, 0)),
        out_shape=jax.ShapeDtypeStruct((M, N), jnp.float32),
    )(x, w, b.reshape(1, N))


def _qproj(q2d, w_off, b_off, w_attn, b_attn, bd):
    M = q2d.shape[0]
    bm = 512

    def body(q_ref, wo_ref, bo_ref, wa_ref, ba_ref, bd_ref, off_ref, aw_ref):
        q = q_ref[...]
        off_ref[...] = (
            jnp.dot(q, wo_ref[...], preferred_element_type=jnp.float32)
            + bo_ref[...]
        )
        lg = (
            jnp.dot(q, wa_ref[...], preferred_element_type=jnp.float32)
            + ba_ref[...]
        )
        m = jnp.exp(lg - jnp.max(lg, axis=-1, keepdims=True))
        denom = jnp.dot(m, bd_ref[...], preferred_element_type=jnp.float32)
        aw_ref[...] = m / denom

    n_off = w_off.shape[1]
    n_at = w_attn.shape[1]
    return pl.pallas_call(
        body,
        grid=(M // bm,),
        in_specs=[
            pl.BlockSpec((bm, D), lambda i: (i, 0)),
            pl.BlockSpec((D, n_off), lambda i: (0, 0)),
            pl.BlockSpec((1, n_off), lambda i: (0, 0)),
            pl.BlockSpec((D, n_at), lambda i: (0, 0)),
            pl.BlockSpec((1, n_at), lambda i: (0, 0)),
            pl.BlockSpec((n_at, n_at), lambda i: (0, 0)),
        ],
        out_specs=[
            pl.BlockSpec((bm, n_off), lambda i: (i, 0)),
            pl.BlockSpec((bm, n_at), lambda i: (i, 0)),
        ],
        out_shape=[
            jax.ShapeDtypeStruct((M, n_off), jnp.float32),
            jax.ShapeDtypeStruct((M, n_at), jnp.float32),
        ],
    )(q2d, w_off, b_off.reshape(1, n_off), w_attn, b_attn.reshape(1, n_at), bd)


def _samp_prep(refx, refy, offx, offy, aw, wf, hf, ibase):
    SR = S // 128
    bm = 512

    def body(rx_ref, ry_ref, ox_ref, oy_ref, aw_ref, wf_ref, hf_ref, ib_ref,
             i00, i10, i01, i11, w00, w10, w01, w11):
        wfv = wf_ref[...]
        hfv = hf_ref[...]
        ix = rx_ref[...] * wfv + ox_ref[...] - 0.5
        iy = ry_ref[...] * hfv + oy_ref[...] - 0.5
        x0f = jnp.floor(ix)
        y0f = jnp.floor(iy)
        wx1 = ix - x0f
        wx0 = 1.0 - wx1
        wy1 = iy - y0f
        wy0 = 1.0 - wy1
        x0 = x0f.astype(jnp.int32)
        y0 = y0f.astype(jnp.int32)
        x1 = x0 + 1
        y1 = y0 + 1
        wi = wfv.astype(jnp.int32)
        hi = hfv.astype(jnp.int32)
        ib = ib_ref[...]
        awv = aw_ref[...]

        def corner(xi, yi, wx, wy, iref, wref):
            valid = (xi >= 0) & (xi < wi) & (yi >= 0) & (yi < hi)
            xc = jnp.clip(xi, 0, wi - 1)
            yc = jnp.clip(yi, 0, hi - 1)
            iref[...] = ib + yc * (wi * 8) + xc * 8
            wref[...] = jnp.where(valid, awv * wx * wy, 0.0)

        corner(x0, y0, wx0, wy0, i00, w00)
        corner(x1, y0, wx1, wy0, i10, w10)
        corner(x0, y1, wx0, wy1, i01, w01)
        corner(x1, y1, wx1, wy1, i11, w11)

    spec = pl.BlockSpec((bm, 128), lambda i: (i, 0))
    f32 = jnp.float32
    return pl.pallas_call(
        body,
        grid=(SR // bm,),
        in_specs=[spec] * 8,
        out_specs=[spec] * 8,
        out_shape=[jax.ShapeDtypeStruct((SR, 128), jnp.int32)] * 4
        + [jax.ShapeDtypeStruct((SR, 128), f32)] * 4,
    )(
        refx.reshape(SR, 128), refy.reshape(SR, 128),
        offx.reshape(SR, 128), offy.reshape(SR, 128),
        aw.reshape(SR, 128), wf.reshape(SR, 128), hf.reshape(SR, 128),
        ibase.reshape(SR, 128),
    )


def _sc_gather_combine(table, idxs, wtss):
    mesh = plsc.VectorSubcoreMesh(
        core_axis_name="c", subcore_axis_name="s", num_cores=2, num_subcores=16
    )
    nc = 2
    cs = CH_SLOTS
    cr = cs * 12

    @functools.partial(
        pl.kernel,
        out_type=jax.ShapeDtypeStruct((NSLOT * HD,), jnp.float32),
        mesh=mesh,
        scratch_types=[
            pltpu.VMEM((2 * 4 * cr,), jnp.int32),
            pltpu.VMEM((2 * 4 * cr,), jnp.float32),
            pltpu.VMEM((2 * 4 * cr, HD), jnp.float32),
            pltpu.VMEM((cs * HD,), jnp.float32),
            pltpu.SemaphoreType.DMA,
            pltpu.SemaphoreType.DMA,
        ],
        compiler_params=pltpu.CompilerParams(use_tc_tiling_on_sc=False),
    )
    def k(table_hbm, i0, i1, i2, i3, v0, v1, v2, v3, out_hbm,
          idx_v, w_v, rows_v, out_v, semi, sem):
        wid = lax.axis_index("s") * nc + lax.axis_index("c")
        slot0 = wid * SLOTS_PER_W
        ihbm = (i0, i1, i2, i3)
        whbm = (v0, v1, v2, v3)
        dn = lax.GatherDimensionNumbers(
            offset_dims=(), collapsed_slice_dims=(0,), start_index_map=(0,)
        )

        def in_copies(ch, par):
            row0 = (slot0 + ch * cs) * 12
            b0 = par * 4 * cr
            cps = []
            for ci in range(4):
                cps.append(pltpu.make_async_copy(
                    ihbm[ci].at[pl.ds(row0, cr)],
                    idx_v.at[pl.ds(b0 + ci * cr, cr)], semi))
                cps.append(pltpu.make_async_copy(
                    whbm[ci].at[pl.ds(row0, cr)],
                    w_v.at[pl.ds(b0 + ci * cr, cr)], semi))
            return cps

        def gathers(par):
            b0 = par * 4 * cr
            return [
                pltpu.make_async_copy(
                    table_hbm.at[idx_v.at[pl.ds(b0 + ci * cr, cr)]],
                    rows_v.at[pl.ds(b0 + ci * cr, cr)],
                    sem,
                )
                for ci in range(4)
            ]

        for cp in in_copies(0, 0):
            cp.start()
        for cp in in_copies(0, 0):
            cp.wait()
        for cp in gathers(0):
            cp.start()
        for cp in in_copies(1, 1):
            cp.start()

        def chunk(c, _):
            p = lax.rem(c, 2)
            q = 1 - p
            slot = slot0 + c * cs
            cn1 = lax.rem(c + 1, N_CHUNK)
            cn2 = lax.rem(c + 2, N_CHUNK)
            for cp in gathers(p):
                cp.wait()
            for cp in in_copies(cn1, q):
                cp.wait()
            for cp in gathers(q):
                cp.start()
            pofs = p * 4 * cr
            accs = [
                [jnp.zeros((16,), jnp.float32), jnp.zeros((16,), jnp.float32)]
                for _ in range(cs)
            ]
            for ci in range(4):
                for grp in range(cr // 16):
                    wgrp = w_v[pl.ds(pofs + ci * cr + grp * 16, 16)]
                    for t in range(16):
                        jj = grp * 16 + t
                        g = jj // 12
                        r = ci * cr + jj
                        wb = lax.gather(
                            wgrp,
                            jnp.full((16, 1), t, jnp.int32),
                            dn,
                            (1,),
                            mode=lax.GatherScatterMode.PROMISE_IN_BOUNDS,
                        )
                        accs[g][0] = accs[g][0] + wb * rows_v[pofs + r, pl.ds(0, 16)]
                        accs[g][1] = accs[g][1] + wb * rows_v[pofs + r, pl.ds(16, 16)]
            for g in range(cs):
                out_v[pl.ds(g * HD, 16)] = accs[g][0]
                out_v[pl.ds(g * HD + 16, 16)] = accs[g][1]
            for cp in in_copies(cn2, p):
                cp.start()
            pltpu.sync_copy(out_v, out_hbm.at[pl.ds(slot * HD, cs * HD)])
            return 0

        lax.fori_loop(0, N_CHUNK, chunk, 0)
        for cp in gathers(0):
            cp.wait()
        for cp in in_copies(1, 1):
            cp.wait()

    return k(table, *idxs, *wtss)


def kernel(query, reference_points, value_0, value_1, value_2, W_value,
           b_value, W_off, b_off, W_attn, b_attn, W_out, b_out):
    f32 = jnp.float32
    v_all = jnp.concatenate([value_0, value_1, value_2], axis=1)
    v_all = v_all.reshape(BB * TOT_POS, D)
    table = _mm_bias(v_all, W_value, b_value, bm=256)
    table = table.reshape(TABLE_ROWS, HD)

    q2d = query.reshape(BQ, D)
    n_at = NH * NL * NP
    seg = jnp.arange(n_at, dtype=jnp.int32) // (NL * NP)
    bd = (seg[:, None] == seg[None, :]).astype(f32)
    off2d, aw2d = _qproj(q2d, W_off, b_off, W_attn, b_attn, bd)

    off = off2d.reshape(BQ, NH, NL, NP, 2)
    offx = off[..., 0].reshape(S)
    offy = off[..., 1].reshape(S)
    aw = aw2d.reshape(S)
    rp = reference_points.reshape(BQ, NL, 2)
    refx = jnp.broadcast_to(rp[:, None, :, None, 0], (BQ, NH, NL, NP)).reshape(S)
    refy = jnp.broadcast_to(rp[:, None, :, None, 1], (BQ, NH, NL, NP)).reshape(S)

    wl = jnp.array([w for (_, w) in SPATIAL], f32)
    hl = jnp.array([h for (h, _) in SPATIAL], f32)
    wf = jnp.broadcast_to(wl[None, None, :, None], (BQ, NH, NL, NP)).reshape(S)
    hf = jnp.broadcast_to(hl[None, None, :, None], (BQ, NH, NL, NP)).reshape(S)
    b_ix = jnp.arange(BQ, dtype=jnp.int32) // NQ
    h_ix = jnp.arange(NH, dtype=jnp.int32)
    base_l = jnp.array(LVL_BASE, jnp.int32)
    ibase = (
        (b_ix[:, None, None] * TOT_POS + base_l[None, None, :]) * 8
        + h_ix[None, :, None]
    )
    ibase = jnp.broadcast_to(ibase[..., None], (BQ, NH, NL, NP)).reshape(S)

    i00, i10, i01, i11, w00, w10, w01, w11 = _samp_prep(
        refx, refy, offx, offy, aw, wf, hf, ibase
    )

    heads = _sc_gather_combine(
        table,
        [x.reshape(S) for x in (i00, i10, i01, i11)],
        [x.reshape(S) for x in (w00, w10, w01, w11)],
    )

    out = _mm_bias(heads.reshape(BQ, D), W_out, b_out, bm=512)
    return out.reshape(BB, NQ, D)
